# Initial kernel scaffold; baseline (speedup 1.0000x reference)
import jax
import jax.numpy as jnp
from jax.experimental import pallas as pl


def kernel(x, wq, bq, wk, bk, wv, bv, gamma, beta):
    raise NotImplementedError("write your pallas kernel here")


# speedup vs baseline: 3.0752x; 3.0752x over previous
import functools
import math

import jax
import jax.numpy as jnp
from jax import lax
from jax.experimental import pallas as pl
from jax.experimental.pallas import tpu as pltpu

NUM_HEADS = 12
LN_EPS = 1e-5


def _mha_ln_kernel(x_ref, wq_ref, bq_ref, wk_ref, bk_ref, wv_ref, bv_ref,
                   g_ref, b_ref, out_ref, *, scale, num_heads):
    xf = x_ref[0]
    xb = xf.astype(jnp.bfloat16)

    q = jnp.dot(xb, wq_ref[...], preferred_element_type=jnp.float32) + bq_ref[...]
    k = jnp.dot(xb, wk_ref[...], preferred_element_type=jnp.float32) + bk_ref[...]
    v = jnp.dot(xb, wv_ref[...], preferred_element_type=jnp.float32) + bv_ref[...]

    S, E = xf.shape
    Dh = E // num_heads

    qb = (q * scale).astype(jnp.bfloat16)
    kb = k.astype(jnp.bfloat16)
    vb = v.astype(jnp.bfloat16)

    ctx_parts = []
    for h in range(num_heads):
        qh = qb[:, h * Dh:(h + 1) * Dh]
        kh = kb[:, h * Dh:(h + 1) * Dh]
        vh = vb[:, h * Dh:(h + 1) * Dh]
        s = lax.dot_general(qh, kh, dimension_numbers=(((1,), (1,)), ((), ())),
                            preferred_element_type=jnp.float32)
        m = jnp.max(s, axis=-1, keepdims=True)
        p = jnp.exp(s - m)
        denom = jnp.sum(p, axis=-1, keepdims=True)
        ctx_h = jnp.dot(p.astype(jnp.bfloat16), vh,
                        preferred_element_type=jnp.float32)
        ctx_parts.append(ctx_h * pl.reciprocal(denom, approx=False))
    ctx = jnp.concatenate(ctx_parts, axis=-1)

    y = 4.0 * xf + ctx
    mean = jnp.mean(y, axis=-1, keepdims=True)
    c = y - mean
    var = jnp.mean(c * c, axis=-1, keepdims=True)
    inv = lax.rsqrt(var + LN_EPS)
    out_ref[0] = (c * inv) * g_ref[...] + b_ref[...]


def kernel(x, wq, bq, wk, bk, wv, bv, gamma, beta):
    B, S, E = x.shape
    scale = 1.0 / math.sqrt(E // NUM_HEADS)

    row_spec = pl.BlockSpec((1, S, E), lambda b: (b, 0, 0))
    w_spec = pl.BlockSpec((E, E), lambda b: (0, 0))
    vec_spec = pl.BlockSpec((1, E), lambda b: (0, 0))

    return pl.pallas_call(
        functools.partial(_mha_ln_kernel, scale=scale, num_heads=NUM_HEADS),
        out_shape=jax.ShapeDtypeStruct((B, S, E), jnp.float32),
        grid=(B,),
        in_specs=[row_spec,
                  w_spec, vec_spec,
                  w_spec, vec_spec,
                  w_spec, vec_spec,
                  vec_spec, vec_spec],
        out_specs=row_spec,
        compiler_params=pltpu.CompilerParams(
            dimension_semantics=("parallel",),
            vmem_limit_bytes=64 * 1024 * 1024,
        ),
    )(x,
      wq.astype(jnp.bfloat16), bq.reshape(1, E),
      wk.astype(jnp.bfloat16), bk.reshape(1, E),
      wv.astype(jnp.bfloat16), bv.reshape(1, E),
      gamma.reshape(1, E), beta.reshape(1, E))
